# baseline (speedup 1.0000x reference)
import functools

import jax
import jax.numpy as jnp
from jax import lax
from jax.experimental import pallas as pl
from jax.experimental.pallas import tpu as pltpu
from jax.experimental.pallas import tpu_sc as plsc

VOCAB = 1000000
DIM = 64
BATCH = 16384
NUM_NEG = 20

NC = 2
NS = 16
NW = NC * NS
PAIRS_W = BATCH // NW
CHUNK = 32
NCHUNK = PAIRS_W // CHUNK
NEG_W = PAIRS_W * NUM_NEG
NEG_CHUNK = CHUNK * NUM_NEG
GROWS = 128
NEG_GATHERS = NEG_CHUNK // GROWS


def _sc_scores(tw, cw, ns, ttab, ctab):
  mesh = plsc.VectorSubcoreMesh(core_axis_name="c", subcore_axis_name="s")

  @functools.partial(
      pl.kernel,
      out_type=(
          jax.ShapeDtypeStruct((BATCH,), jnp.float32),
          jax.ShapeDtypeStruct((BATCH * NUM_NEG,), jnp.float32),
      ),
      mesh=mesh,
      scratch_types=[
          pltpu.VMEM((PAIRS_W,), jnp.int32),
          pltpu.VMEM((PAIRS_W,), jnp.int32),
          pltpu.VMEM((NEG_W,), jnp.int32),
          pltpu.VMEM((CHUNK, DIM), jnp.float32),
          pltpu.VMEM((CHUNK, DIM), jnp.float32),
          pltpu.VMEM((NEG_CHUNK, DIM), jnp.float32),
          pltpu.VMEM((CHUNK, DIM), jnp.float32),
          pltpu.VMEM((CHUNK, DIM), jnp.float32),
          pltpu.VMEM((NEG_CHUNK, DIM), jnp.float32),
          pltpu.VMEM((PAIRS_W,), jnp.float32),
          pltpu.VMEM((NEG_W,), jnp.float32),
          pltpu.SemaphoreType.DMA,
          pltpu.SemaphoreType.DMA,
      ],
      compiler_params=pltpu.CompilerParams(
          needs_layout_passes=False, use_tc_tiling_on_sc=False),
  )
  def k(tw_hbm, cw_hbm, ns_hbm, ttab_hbm, ctab_hbm, pos_hbm, neg_hbm,
        tidx, cidx, nidx, trowsA, crowsA, nrowsA, trowsB, crowsB, nrowsB,
        posv, negv, semA, semB):
    wid = lax.axis_index("s") * NC + lax.axis_index("c")
    pbase = wid * PAIRS_W
    nbase = wid * NEG_W
    pltpu.sync_copy(tw_hbm.at[pl.ds(pbase, PAIRS_W)], tidx)
    pltpu.sync_copy(cw_hbm.at[pl.ds(pbase, PAIRS_W)], cidx)
    pltpu.sync_copy(ns_hbm.at[pl.ds(nbase, NEG_W)], nidx)

    def remap(ref, n16):
      def rbody(i, c):
        r = ref[pl.ds(i * 16, 16)]
        u = r & (RBLK - 1)
        base = (r - u) + 2 * u
        ref[pl.ds(i * 16, 16)] = jnp.where(
            u >= RBLK // 2, base - (RBLK - 1), base)
        return c
      lax.fori_loop(0, n16, rbody, 0)

    remap(tidx, PAIRS_W // 16)
    remap(cidx, PAIRS_W // 16)
    remap(nidx, NEG_W // 16)

    lane = lax.iota(jnp.int32, 16)

    def issue(ci, trows, crows, nrows, sem):
      cp = ci * CHUNK
      pltpu.async_copy(ttab_hbm.at[tidx.at[pl.ds(cp, CHUNK)]], trows, sem)
      pltpu.async_copy(ctab_hbm.at[cidx.at[pl.ds(cp, CHUNK)]], crows, sem)
      for g in range(NEG_GATHERS):
        pltpu.async_copy(
            ttab_hbm.at[nidx.at[pl.ds(ci * NEG_CHUNK + g * GROWS, GROWS)]],
            nrows.at[pl.ds(g * GROWS, GROWS)], sem)

    def drain(trows, crows, nrows, sem):
      pltpu.make_async_copy(ttab_hbm.at[tidx.at[pl.ds(0, CHUNK)]],
                            trows, sem).wait()
      pltpu.make_async_copy(ctab_hbm.at[cidx.at[pl.ds(0, CHUNK)]],
                            crows, sem).wait()
      for g in range(NEG_GATHERS):
        pltpu.make_async_copy(
            ttab_hbm.at[nidx.at[pl.ds(g * GROWS, GROWS)]],
            nrows.at[pl.ds(g * GROWS, GROWS)], sem).wait()

    def compute(ci, trows, crows, nrows):
      cp = ci * CHUNK
      for grp in range(2):
        p0 = grp * 16
        prow = p0 + lane
        nrow0 = prow * NUM_NEG

        def dbody(d, accs):
          dcol = (d + lane) & (DIM - 1)
          cv = plsc.load_gather(crows, [prow, dcol])
          tv = plsc.load_gather(trows, [prow, dcol])
          outs = [accs[0] + cv * tv]
          for kk in range(NUM_NEG):
            nv = plsc.load_gather(nrows, [nrow0 + kk, dcol])
            outs.append(accs[kk + 1] + nv * cv)
          return tuple(outs)

        accs = lax.fori_loop(
            0, DIM, dbody,
            tuple(jnp.zeros((16,), jnp.float32) for _ in range(NUM_NEG + 1)))
        posv[pl.ds(cp + p0, 16)] = accs[0]
        sbase = (ci * 2 + grp) * (16 * NUM_NEG)
        for kk in range(NUM_NEG):
          negv[pl.ds(sbase + kk * 16, 16)] = -accs[kk + 1]

    issue(0, trowsA, crowsA, nrowsA, semA)

    def pair_body(ci2, carry):
      ca = ci2 * 2
      issue(ca + 1, trowsB, crowsB, nrowsB, semB)
      drain(trowsA, crowsA, nrowsA, semA)
      compute(ca, trowsA, crowsA, nrowsA)

      @pl.when(ci2 < NCHUNK // 2 - 1)
      def _():
        issue(ca + 2, trowsA, crowsA, nrowsA, semA)

      drain(trowsB, crowsB, nrowsB, semB)
      compute(ca + 1, trowsB, crowsB, nrowsB)
      return carry

    lax.fori_loop(0, NCHUNK // 2, pair_body, 0)
    pltpu.sync_copy(posv, pos_hbm.at[pl.ds(pbase, PAIRS_W)])
    pltpu.sync_copy(negv, neg_hbm.at[pl.ds(nbase, NEG_W)])

  return k(tw, cw, ns, ttab, ctab)


RBLK = 1024
NRBLK = (VOCAB + RBLK - 1) // RBLK
VROWS = NRBLK * RBLK


def _transpose_body(in_ref, out_ref):
  x = in_ref[...]
  eye = jnp.eye(DIM, dtype=jnp.float32)
  y = jax.lax.dot_general(x, eye, (((0,), (0,)), ((), ())),
                          preferred_element_type=jnp.float32)
  out_ref[:, 0:DIM] = y[0:RBLK // 2, :]
  out_ref[:, DIM:2 * DIM] = y[RBLK // 2:RBLK, :]


def _relayout(table):
  tv = jnp.swapaxes(table, 0, 1)
  out = pl.pallas_call(
      _transpose_body,
      grid=(NRBLK,),
      in_specs=[pl.BlockSpec((DIM, RBLK), lambda i: (0, i))],
      out_specs=pl.BlockSpec((RBLK // 2, 2 * DIM), lambda i: (i, 0)),
      out_shape=jax.ShapeDtypeStruct((VROWS // 2, 2 * DIM), jnp.float32),
  )(tv)
  return out.reshape(VROWS, DIM)


def _loss_body(pos_ref, neg_ref, out_ref):
  s = jnp.sum(jax.nn.log_sigmoid(pos_ref[...]))
  s = s + jnp.sum(jax.nn.log_sigmoid(neg_ref[...]))
  out_ref[0, 0] = -(s / BATCH)


def kernel(target_words, context_words, negative_samples, target_table,
           context_table):
  tw = target_words.astype(jnp.int32)
  cw = context_words.astype(jnp.int32)
  ns = negative_samples.astype(jnp.int32).reshape(-1)
  pos, neg = _sc_scores(tw, cw, ns, _relayout(target_table),
                        _relayout(context_table))
  out = pl.pallas_call(
      _loss_body,
      out_shape=jax.ShapeDtypeStruct((1, 1), jnp.float32),
      out_specs=pl.BlockSpec(memory_space=pltpu.SMEM),
  )(pos.reshape(128, 128), neg.reshape(BATCH * NUM_NEG // 128, 128))
  return out[0, 0]


# speedup vs baseline: 2.0518x; 2.0518x over previous
import functools

import jax
import jax.numpy as jnp
from jax import lax
from jax.experimental import pallas as pl
from jax.experimental.pallas import tpu as pltpu
from jax.experimental.pallas import tpu_sc as plsc

VOCAB = 1000000
DIM = 64
BATCH = 16384
NUM_NEG = 20

NC = 2
NS = 16
NW = NC * NS
PAIRS_W = BATCH // NW
CHUNK = 32
NCHUNK = PAIRS_W // CHUNK
NEG_W = PAIRS_W * NUM_NEG
NEG_CHUNK = CHUNK * NUM_NEG
GROWS = 128
NEG_GATHERS = NEG_CHUNK // GROWS


def _sc_scores(tw, cw, ns, ttab, ctab):
  mesh = plsc.VectorSubcoreMesh(core_axis_name="c", subcore_axis_name="s")

  @functools.partial(
      pl.kernel,
      out_type=(
          jax.ShapeDtypeStruct((BATCH,), jnp.float32),
          jax.ShapeDtypeStruct((BATCH * NUM_NEG,), jnp.float32),
      ),
      mesh=mesh,
      scratch_types=[
          pltpu.VMEM((PAIRS_W,), jnp.int32),
          pltpu.VMEM((PAIRS_W,), jnp.int32),
          pltpu.VMEM((NEG_W,), jnp.int32),
          pltpu.VMEM((CHUNK, DIM), jnp.float32),
          pltpu.VMEM((CHUNK, DIM), jnp.float32),
          pltpu.VMEM((NEG_CHUNK, DIM), jnp.float32),
          pltpu.VMEM((CHUNK, DIM), jnp.float32),
          pltpu.VMEM((CHUNK, DIM), jnp.float32),
          pltpu.VMEM((NEG_CHUNK, DIM), jnp.float32),
          pltpu.VMEM((PAIRS_W,), jnp.float32),
          pltpu.VMEM((NEG_W,), jnp.float32),
          pltpu.SemaphoreType.DMA,
          pltpu.SemaphoreType.DMA,
      ],
      compiler_params=pltpu.CompilerParams(
          needs_layout_passes=False, use_tc_tiling_on_sc=False),
  )
  def k(tw_hbm, cw_hbm, ns_hbm, ttab_hbm, ctab_hbm, pos_hbm, neg_hbm,
        tidx, cidx, nidx, trowsA, crowsA, nrowsA, trowsB, crowsB, nrowsB,
        posv, negv, semA, semB):
    wid = lax.axis_index("s") * NC + lax.axis_index("c")
    pbase = wid * PAIRS_W
    nbase = wid * NEG_W
    pltpu.sync_copy(tw_hbm.at[pl.ds(pbase, PAIRS_W)], tidx)
    pltpu.sync_copy(cw_hbm.at[pl.ds(pbase, PAIRS_W)], cidx)
    pltpu.sync_copy(ns_hbm.at[pl.ds(nbase, NEG_W)], nidx)

    def remap(ref, n16):
      def rbody(i, c):
        r = ref[pl.ds(i * 16, 16)]
        u = r & (RBLK - 1)
        base = (r - u) + 2 * u
        ref[pl.ds(i * 16, 16)] = jnp.where(
            u >= RBLK // 2, base - (RBLK - 1), base)
        return c
      lax.fori_loop(0, n16, rbody, 0)

    remap(tidx, PAIRS_W // 16)
    remap(cidx, PAIRS_W // 16)
    remap(nidx, NEG_W // 16)

    lane = lax.iota(jnp.int32, 16)

    def issue(ci, trows, crows, nrows, sem):
      cp = ci * CHUNK
      pltpu.async_copy(ttab_hbm.at[tidx.at[pl.ds(cp, CHUNK)]], trows, sem)
      pltpu.async_copy(ctab_hbm.at[cidx.at[pl.ds(cp, CHUNK)]], crows, sem)
      for g in range(NEG_GATHERS):
        pltpu.async_copy(
            ttab_hbm.at[nidx.at[pl.ds(ci * NEG_CHUNK + g * GROWS, GROWS)]],
            nrows.at[pl.ds(g * GROWS, GROWS)], sem)

    def drain(trows, crows, nrows, sem):
      pltpu.make_async_copy(ttab_hbm.at[tidx.at[pl.ds(0, CHUNK)]],
                            trows, sem).wait()
      pltpu.make_async_copy(ctab_hbm.at[cidx.at[pl.ds(0, CHUNK)]],
                            crows, sem).wait()
      for g in range(NEG_GATHERS):
        pltpu.make_async_copy(
            ttab_hbm.at[nidx.at[pl.ds(g * GROWS, GROWS)]],
            nrows.at[pl.ds(g * GROWS, GROWS)], sem).wait()

    def compute(ci, trows, crows, nrows):
      cp = ci * CHUNK
      for grp in range(2):
        p0 = grp * 16
        prow = p0 + lane
        nrow0 = prow * NUM_NEG

        def dbody(d, accs):
          dcol = (d + lane) & (DIM - 1)
          cv = plsc.load_gather(crows, [prow, dcol])
          tv = plsc.load_gather(trows, [prow, dcol])
          outs = [accs[0] + cv * tv]
          for kk in range(NUM_NEG):
            nv = plsc.load_gather(nrows, [nrow0 + kk, dcol])
            outs.append(accs[kk + 1] + nv * cv)
          return tuple(outs)

        accs = lax.fori_loop(
            0, DIM, dbody,
            tuple(jnp.zeros((16,), jnp.float32) for _ in range(NUM_NEG + 1)))
        posv[pl.ds(cp + p0, 16)] = accs[0]
        sbase = (ci * 2 + grp) * (16 * NUM_NEG)
        for kk in range(NUM_NEG):
          negv[pl.ds(sbase + kk * 16, 16)] = -accs[kk + 1]

    issue(0, trowsA, crowsA, nrowsA, semA)

    def pair_body(ci2, carry):
      ca = ci2 * 2
      issue(ca + 1, trowsB, crowsB, nrowsB, semB)
      drain(trowsA, crowsA, nrowsA, semA)
      compute(ca, trowsA, crowsA, nrowsA)

      @pl.when(ci2 < NCHUNK // 2 - 1)
      def _():
        issue(ca + 2, trowsA, crowsA, nrowsA, semA)

      drain(trowsB, crowsB, nrowsB, semB)
      compute(ca + 1, trowsB, crowsB, nrowsB)
      return carry

    lax.fori_loop(0, NCHUNK // 2, pair_body, 0)
    pltpu.sync_copy(posv, pos_hbm.at[pl.ds(pbase, PAIRS_W)])
    pltpu.sync_copy(negv, neg_hbm.at[pl.ds(nbase, NEG_W)])

  return k(tw, cw, ns, ttab, ctab)


RBLK = 4096
NRBLK = (VOCAB + RBLK - 1) // RBLK
VROWS = NRBLK * RBLK


def _transpose_body(in_ref, out_ref):
  x = in_ref[...]
  eye = jnp.eye(DIM, dtype=jnp.float32)
  y = jax.lax.dot_general(x, eye, (((0,), (0,)), ((), ())),
                          preferred_element_type=jnp.float32)
  out_ref[:, 0:DIM] = y[0:RBLK // 2, :]
  out_ref[:, DIM:2 * DIM] = y[RBLK // 2:RBLK, :]


def _relayout(table):
  tv = jnp.swapaxes(table, 0, 1)
  out = pl.pallas_call(
      _transpose_body,
      grid=(NRBLK,),
      in_specs=[pl.BlockSpec((DIM, RBLK), lambda i: (0, i))],
      out_specs=pl.BlockSpec((RBLK // 2, 2 * DIM), lambda i: (i, 0)),
      out_shape=jax.ShapeDtypeStruct((VROWS // 2, 2 * DIM), jnp.float32),
  )(tv)
  return out.reshape(VROWS, DIM)


def _loss_body(pos_ref, neg_ref, out_ref):
  s = jnp.sum(jax.nn.log_sigmoid(pos_ref[...]))
  s = s + jnp.sum(jax.nn.log_sigmoid(neg_ref[...]))
  out_ref[0, 0] = -(s / BATCH)


def kernel(target_words, context_words, negative_samples, target_table,
           context_table):
  tw = target_words.astype(jnp.int32)
  cw = context_words.astype(jnp.int32)
  ns = negative_samples.astype(jnp.int32).reshape(-1)
  pos, neg = _sc_scores(tw, cw, ns, _relayout(target_table),
                        _relayout(context_table))
  out = pl.pallas_call(
      _loss_body,
      out_shape=jax.ShapeDtypeStruct((1, 1), jnp.float32),
      out_specs=pl.BlockSpec(memory_space=pltpu.SMEM),
  )(pos.reshape(128, 128), neg.reshape(BATCH * NUM_NEG // 128, 128))
  return out[0, 0]


# speedup vs baseline: 2.8038x; 1.3665x over previous
import functools

import jax
import jax.numpy as jnp
from jax import lax
from jax.experimental import pallas as pl
from jax.experimental.pallas import tpu as pltpu
from jax.experimental.pallas import tpu_sc as plsc

VOCAB = 1000000
DIM = 64
BATCH = 16384
NUM_NEG = 20

NC = 2
NS = 16
NW = NC * NS
PAIRS_W = BATCH // NW
CHUNK = 32
NCHUNK = PAIRS_W // CHUNK
NEG_W = PAIRS_W * NUM_NEG
NEG_CHUNK = CHUNK * NUM_NEG
GROWS = 128
NEG_GATHERS = NEG_CHUNK // GROWS


def _sc_scores(tw, cw, ns, ttab, ctab):
  mesh = plsc.VectorSubcoreMesh(core_axis_name="c", subcore_axis_name="s")

  @functools.partial(
      pl.kernel,
      out_type=(
          jax.ShapeDtypeStruct((BATCH,), jnp.float32),
          jax.ShapeDtypeStruct((BATCH * NUM_NEG,), jnp.float32),
      ),
      mesh=mesh,
      scratch_types=[
          pltpu.VMEM((PAIRS_W,), jnp.int32),
          pltpu.VMEM((PAIRS_W,), jnp.int32),
          pltpu.VMEM((NEG_W,), jnp.int32),
          pltpu.VMEM((CHUNK, DIM), jnp.float32),
          pltpu.VMEM((CHUNK, DIM), jnp.float32),
          pltpu.VMEM((NEG_CHUNK, DIM), jnp.float32),
          pltpu.VMEM((CHUNK, DIM), jnp.float32),
          pltpu.VMEM((CHUNK, DIM), jnp.float32),
          pltpu.VMEM((NEG_CHUNK, DIM), jnp.float32),
          pltpu.VMEM((PAIRS_W,), jnp.float32),
          pltpu.VMEM((NEG_W,), jnp.float32),
          pltpu.SemaphoreType.DMA,
          pltpu.SemaphoreType.DMA,
      ],
      compiler_params=pltpu.CompilerParams(
          needs_layout_passes=False, use_tc_tiling_on_sc=False),
  )
  def k(tw_hbm, cw_hbm, ns_hbm, ttab_hbm, ctab_hbm, pos_hbm, neg_hbm,
        tidx, cidx, nidx, trowsA, crowsA, nrowsA, trowsB, crowsB, nrowsB,
        posv, negv, semA, semB):
    wid = lax.axis_index("s") * NC + lax.axis_index("c")
    pbase = wid * PAIRS_W
    nbase = wid * NEG_W
    pltpu.sync_copy(tw_hbm.at[pl.ds(pbase, PAIRS_W)], tidx)
    pltpu.sync_copy(cw_hbm.at[pl.ds(pbase, PAIRS_W)], cidx)
    pltpu.sync_copy(ns_hbm.at[pl.ds(nbase, NEG_W)], nidx)

    def remap(ref, n16):
      def rbody(i, c):
        r = ref[pl.ds(i * 16, 16)]
        u = r & (RBLK - 1)
        base = (r - u) + 2 * u
        ref[pl.ds(i * 16, 16)] = jnp.where(
            u >= RBLK // 2, base - (RBLK - 1), base)
        return c
      lax.fori_loop(0, n16, rbody, 0)

    remap(tidx, PAIRS_W // 16)
    remap(cidx, PAIRS_W // 16)
    remap(nidx, NEG_W // 16)

    lane = lax.iota(jnp.int32, 16)

    def issue(ci, trows, crows, nrows, sem):
      cp = ci * CHUNK
      pltpu.async_copy(ttab_hbm.at[tidx.at[pl.ds(cp, CHUNK)]], trows, sem)
      pltpu.async_copy(ctab_hbm.at[cidx.at[pl.ds(cp, CHUNK)]], crows, sem)
      for g in range(NEG_GATHERS):
        pltpu.async_copy(
            ttab_hbm.at[nidx.at[pl.ds(ci * NEG_CHUNK + g * GROWS, GROWS)]],
            nrows.at[pl.ds(g * GROWS, GROWS)], sem)

    def drain(trows, crows, nrows, sem):
      pltpu.make_async_copy(ttab_hbm.at[tidx.at[pl.ds(0, CHUNK)]],
                            trows, sem).wait()
      pltpu.make_async_copy(ctab_hbm.at[cidx.at[pl.ds(0, CHUNK)]],
                            crows, sem).wait()
      for g in range(NEG_GATHERS):
        pltpu.make_async_copy(
            ttab_hbm.at[nidx.at[pl.ds(g * GROWS, GROWS)]],
            nrows.at[pl.ds(g * GROWS, GROWS)], sem).wait()

    def compute(ci, trows, crows, nrows):
      cp = ci * CHUNK
      for grp in range(2):
        p0 = grp * 16
        prow = p0 + lane
        nrow0 = prow * NUM_NEG

        def dbody(d, accs):
          dcol = (d + lane) & (DIM - 1)
          cv = plsc.load_gather(crows, [prow, dcol])
          tv = plsc.load_gather(trows, [prow, dcol])
          outs = [accs[0] + cv * tv]
          for kk in range(NUM_NEG):
            nv = plsc.load_gather(nrows, [nrow0 + kk, dcol])
            outs.append(accs[kk + 1] + nv * cv)
          return tuple(outs)

        accs = lax.fori_loop(
            0, DIM, dbody,
            tuple(jnp.zeros((16,), jnp.float32) for _ in range(NUM_NEG + 1)))
        posv[pl.ds(cp + p0, 16)] = accs[0]
        sbase = (ci * 2 + grp) * (16 * NUM_NEG)
        for kk in range(NUM_NEG):
          negv[pl.ds(sbase + kk * 16, 16)] = -accs[kk + 1]

    issue(0, trowsA, crowsA, nrowsA, semA)

    def pair_body(ci2, carry):
      ca = ci2 * 2
      issue(ca + 1, trowsB, crowsB, nrowsB, semB)
      drain(trowsA, crowsA, nrowsA, semA)
      compute(ca, trowsA, crowsA, nrowsA)

      @pl.when(ci2 < NCHUNK // 2 - 1)
      def _():
        issue(ca + 2, trowsA, crowsA, nrowsA, semA)

      drain(trowsB, crowsB, nrowsB, semB)
      compute(ca + 1, trowsB, crowsB, nrowsB)
      return carry

    lax.fori_loop(0, NCHUNK // 2, pair_body, 0)
    pltpu.sync_copy(posv, pos_hbm.at[pl.ds(pbase, PAIRS_W)])
    pltpu.sync_copy(negv, neg_hbm.at[pl.ds(nbase, NEG_W)])

  return k(tw, cw, ns, ttab, ctab)


RBLK = 16384
NRBLK = (VOCAB + RBLK - 1) // RBLK
VROWS = NRBLK * RBLK


def _transpose_body(in_ref, out_ref):
  x = in_ref[...]
  eye = jnp.eye(DIM, dtype=jnp.float32)
  y = jax.lax.dot_general(x, eye, (((0,), (0,)), ((), ())),
                          preferred_element_type=jnp.float32)
  out_ref[:, 0:DIM] = y[0:RBLK // 2, :]
  out_ref[:, DIM:2 * DIM] = y[RBLK // 2:RBLK, :]


def _relayout(table):
  tv = jnp.swapaxes(table, 0, 1)
  out = pl.pallas_call(
      _transpose_body,
      grid=(NRBLK,),
      in_specs=[pl.BlockSpec((DIM, RBLK), lambda i: (0, i))],
      out_specs=pl.BlockSpec((RBLK // 2, 2 * DIM), lambda i: (i, 0)),
      out_shape=jax.ShapeDtypeStruct((VROWS // 2, 2 * DIM), jnp.float32),
  )(tv)
  return out.reshape(VROWS, DIM)


def _loss_body(pos_ref, neg_ref, out_ref):
  s = jnp.sum(jax.nn.log_sigmoid(pos_ref[...]))
  s = s + jnp.sum(jax.nn.log_sigmoid(neg_ref[...]))
  out_ref[0, 0] = -(s / BATCH)


def kernel(target_words, context_words, negative_samples, target_table,
           context_table):
  tw = target_words.astype(jnp.int32)
  cw = context_words.astype(jnp.int32)
  ns = negative_samples.astype(jnp.int32).reshape(-1)
  pos, neg = _sc_scores(tw, cw, ns, _relayout(target_table),
                        _relayout(context_table))
  out = pl.pallas_call(
      _loss_body,
      out_shape=jax.ShapeDtypeStruct((1, 1), jnp.float32),
      out_specs=pl.BlockSpec(memory_space=pltpu.SMEM),
  )(pos.reshape(128, 128), neg.reshape(BATCH * NUM_NEG // 128, 128))
  return out[0, 0]


# speedup vs baseline: 2.9558x; 1.0542x over previous
import functools

import jax
import jax.numpy as jnp
from jax import lax
from jax.experimental import pallas as pl
from jax.experimental.pallas import tpu as pltpu
from jax.experimental.pallas import tpu_sc as plsc

VOCAB = 1000000
DIM = 64
BATCH = 16384
NUM_NEG = 20

NC = 2
NS = 16
NW = NC * NS
PAIRS_W = BATCH // NW
CHUNK = 32
NCHUNK = PAIRS_W // CHUNK
NEG_W = PAIRS_W * NUM_NEG
NEG_CHUNK = CHUNK * NUM_NEG
GROWS = 128
NEG_GATHERS = NEG_CHUNK // GROWS


def _sc_scores(tw, cw, ns, ttab, ctab):
  mesh = plsc.VectorSubcoreMesh(core_axis_name="c", subcore_axis_name="s")

  @functools.partial(
      pl.kernel,
      out_type=(
          jax.ShapeDtypeStruct((BATCH,), jnp.float32),
          jax.ShapeDtypeStruct((BATCH * NUM_NEG,), jnp.float32),
      ),
      mesh=mesh,
      scratch_types=[
          pltpu.VMEM((PAIRS_W,), jnp.int32),
          pltpu.VMEM((PAIRS_W,), jnp.int32),
          pltpu.VMEM((NEG_W,), jnp.int32),
          pltpu.VMEM((CHUNK, DIM), jnp.float32),
          pltpu.VMEM((CHUNK, DIM), jnp.float32),
          pltpu.VMEM((NEG_CHUNK, DIM), jnp.float32),
          pltpu.VMEM((CHUNK, DIM), jnp.float32),
          pltpu.VMEM((CHUNK, DIM), jnp.float32),
          pltpu.VMEM((NEG_CHUNK, DIM), jnp.float32),
          pltpu.VMEM((PAIRS_W,), jnp.float32),
          pltpu.VMEM((NEG_W,), jnp.float32),
          pltpu.SemaphoreType.DMA,
          pltpu.SemaphoreType.DMA,
      ],
      compiler_params=pltpu.CompilerParams(
          needs_layout_passes=False, use_tc_tiling_on_sc=False),
  )
  def k(tw_hbm, cw_hbm, ns_hbm, ttab_hbm, ctab_hbm, pos_hbm, neg_hbm,
        tidx, cidx, nidx, trowsA, crowsA, nrowsA, trowsB, crowsB, nrowsB,
        posv, negv, semA, semB):
    wid = lax.axis_index("s") * NC + lax.axis_index("c")
    pbase = wid * PAIRS_W
    nbase = wid * NEG_W
    pltpu.sync_copy(tw_hbm.at[pl.ds(pbase, PAIRS_W)], tidx)
    pltpu.sync_copy(cw_hbm.at[pl.ds(pbase, PAIRS_W)], cidx)
    pltpu.sync_copy(ns_hbm.at[pl.ds(nbase, NEG_W)], nidx)

    def remap(ref, n16):
      def rbody(i, c):
        r = ref[pl.ds(i * 16, 16)]
        u = r & (RBLK - 1)
        base = (r - u) + 2 * u
        ref[pl.ds(i * 16, 16)] = jnp.where(
            u >= RBLK // 2, base - (RBLK - 1), base)
        return c
      lax.fori_loop(0, n16, rbody, 0)

    remap(tidx, PAIRS_W // 16)
    remap(cidx, PAIRS_W // 16)
    remap(nidx, NEG_W // 16)

    lane = lax.iota(jnp.int32, 16)

    def issue(ci, trows, crows, nrows, sem):
      cp = ci * CHUNK
      pltpu.async_copy(ttab_hbm.at[tidx.at[pl.ds(cp, CHUNK)]], trows, sem)
      pltpu.async_copy(ctab_hbm.at[cidx.at[pl.ds(cp, CHUNK)]], crows, sem)
      for g in range(NEG_GATHERS):
        pltpu.async_copy(
            ttab_hbm.at[nidx.at[pl.ds(ci * NEG_CHUNK + g * GROWS, GROWS)]],
            nrows.at[pl.ds(g * GROWS, GROWS)], sem)

    def drain(trows, crows, nrows, sem):
      pltpu.make_async_copy(ttab_hbm.at[tidx.at[pl.ds(0, CHUNK)]],
                            trows, sem).wait()
      pltpu.make_async_copy(ctab_hbm.at[cidx.at[pl.ds(0, CHUNK)]],
                            crows, sem).wait()
      for g in range(NEG_GATHERS):
        pltpu.make_async_copy(
            ttab_hbm.at[nidx.at[pl.ds(g * GROWS, GROWS)]],
            nrows.at[pl.ds(g * GROWS, GROWS)], sem).wait()

    def compute(ci, trows, crows, nrows):
      cp = ci * CHUNK
      for grp in range(2):
        p0 = grp * 16
        prow = p0 + lane
        nrow0 = prow * NUM_NEG

        def dbody(d, accs):
          dcol = (d + lane) & (DIM - 1)
          cv = plsc.load_gather(crows, [prow, dcol])
          tv = plsc.load_gather(trows, [prow, dcol])
          outs = [accs[0] + cv * tv]
          for kk in range(NUM_NEG):
            nv = plsc.load_gather(nrows, [nrow0 + kk, dcol])
            outs.append(accs[kk + 1] + nv * cv)
          return tuple(outs)

        accs = lax.fori_loop(
            0, DIM, dbody,
            tuple(jnp.zeros((16,), jnp.float32) for _ in range(NUM_NEG + 1)))
        posv[pl.ds(cp + p0, 16)] = accs[0]
        sbase = (ci * 2 + grp) * (16 * NUM_NEG)
        for kk in range(NUM_NEG):
          negv[pl.ds(sbase + kk * 16, 16)] = -accs[kk + 1]

    issue(0, trowsA, crowsA, nrowsA, semA)

    def pair_body(ci2, carry):
      ca = ci2 * 2
      issue(ca + 1, trowsB, crowsB, nrowsB, semB)
      drain(trowsA, crowsA, nrowsA, semA)
      compute(ca, trowsA, crowsA, nrowsA)

      @pl.when(ci2 < NCHUNK // 2 - 1)
      def _():
        issue(ca + 2, trowsA, crowsA, nrowsA, semA)

      drain(trowsB, crowsB, nrowsB, semB)
      compute(ca + 1, trowsB, crowsB, nrowsB)
      return carry

    lax.fori_loop(0, NCHUNK // 2, pair_body, 0)
    pltpu.sync_copy(posv, pos_hbm.at[pl.ds(pbase, PAIRS_W)])
    pltpu.sync_copy(negv, neg_hbm.at[pl.ds(nbase, NEG_W)])

  return k(tw, cw, ns, ttab, ctab)


RBLK = 32768
NRBLK = (VOCAB + RBLK - 1) // RBLK
VROWS = NRBLK * RBLK


def _transpose_body(in_ref, out_ref):
  x = in_ref[...]
  eye = jnp.eye(DIM, dtype=jnp.float32)
  y = jax.lax.dot_general(x, eye, (((0,), (0,)), ((), ())),
                          preferred_element_type=jnp.float32)
  out_ref[:, 0:DIM] = y[0:RBLK // 2, :]
  out_ref[:, DIM:2 * DIM] = y[RBLK // 2:RBLK, :]


def _relayout(table):
  tv = jnp.swapaxes(table, 0, 1)
  out = pl.pallas_call(
      _transpose_body,
      grid=(NRBLK,),
      in_specs=[pl.BlockSpec((DIM, RBLK), lambda i: (0, i))],
      out_specs=pl.BlockSpec((RBLK // 2, 2 * DIM), lambda i: (i, 0)),
      out_shape=jax.ShapeDtypeStruct((VROWS // 2, 2 * DIM), jnp.float32),
  )(tv)
  return out.reshape(VROWS, DIM)


def _loss_body(pos_ref, neg_ref, out_ref):
  s = jnp.sum(jax.nn.log_sigmoid(pos_ref[...]))
  s = s + jnp.sum(jax.nn.log_sigmoid(neg_ref[...]))
  out_ref[0, 0] = -(s / BATCH)


def kernel(target_words, context_words, negative_samples, target_table,
           context_table):
  tw = target_words.astype(jnp.int32)
  cw = context_words.astype(jnp.int32)
  ns = negative_samples.astype(jnp.int32).reshape(-1)
  pos, neg = _sc_scores(tw, cw, ns, _relayout(target_table),
                        _relayout(context_table))
  out = pl.pallas_call(
      _loss_body,
      out_shape=jax.ShapeDtypeStruct((1, 1), jnp.float32),
      out_specs=pl.BlockSpec(memory_space=pltpu.SMEM),
  )(pos.reshape(128, 128), neg.reshape(BATCH * NUM_NEG // 128, 128))
  return out[0, 0]


# speedup vs baseline: 3.6613x; 1.2387x over previous
import functools

import jax
import jax.numpy as jnp
from jax import lax
from jax.experimental import pallas as pl
from jax.experimental.pallas import tpu as pltpu
from jax.experimental.pallas import tpu_sc as plsc

VOCAB = 1000000
DIM = 64
BATCH = 16384
NUM_NEG = 20

NC = 2
NS = 16
NW = NC * NS
PAIRS_W = BATCH // NW
CHUNK = 32
NCHUNK = PAIRS_W // CHUNK
NEG_W = PAIRS_W * NUM_NEG
NEG_CHUNK = CHUNK * NUM_NEG
GROWS = 128
NEG_GATHERS = NEG_CHUNK // GROWS


def _sc_scores(tw, cw, ns, ttab, ctab):
  mesh = plsc.VectorSubcoreMesh(core_axis_name="c", subcore_axis_name="s")

  @functools.partial(
      pl.kernel,
      out_type=(
          jax.ShapeDtypeStruct((BATCH,), jnp.float32),
          jax.ShapeDtypeStruct((BATCH * NUM_NEG,), jnp.float32),
      ),
      mesh=mesh,
      scratch_types=[
          pltpu.VMEM((PAIRS_W,), jnp.int32),
          pltpu.VMEM((PAIRS_W,), jnp.int32),
          pltpu.VMEM((NEG_W,), jnp.int32),
          pltpu.VMEM((CHUNK, DIM), jnp.float32),
          pltpu.VMEM((CHUNK, DIM), jnp.float32),
          pltpu.VMEM((NEG_CHUNK, DIM), jnp.float32),
          pltpu.VMEM((CHUNK, DIM), jnp.float32),
          pltpu.VMEM((CHUNK, DIM), jnp.float32),
          pltpu.VMEM((NEG_CHUNK, DIM), jnp.float32),
          pltpu.VMEM((PAIRS_W,), jnp.float32),
          pltpu.VMEM((NEG_W,), jnp.float32),
          pltpu.SemaphoreType.DMA,
          pltpu.SemaphoreType.DMA,
      ],
      compiler_params=pltpu.CompilerParams(
          needs_layout_passes=False, use_tc_tiling_on_sc=False),
  )
  def k(tw_hbm, cw_hbm, ns_hbm, ttab_hbm, ctab_hbm, pos_hbm, neg_hbm,
        tidx, cidx, nidx, trowsA, crowsA, nrowsA, trowsB, crowsB, nrowsB,
        posv, negv, semA, semB):
    wid = lax.axis_index("s") * NC + lax.axis_index("c")
    pbase = wid * PAIRS_W
    nbase = wid * NEG_W
    pltpu.sync_copy(tw_hbm.at[pl.ds(pbase, PAIRS_W)], tidx)
    pltpu.sync_copy(cw_hbm.at[pl.ds(pbase, PAIRS_W)], cidx)
    pltpu.sync_copy(ns_hbm.at[pl.ds(nbase, NEG_W)], nidx)

    def remap(ref, n16):
      def rbody(i, c):
        r = ref[pl.ds(i * 16, 16)]
        u = r & (RBLK - 1)
        base = (r - u) + 2 * u
        ref[pl.ds(i * 16, 16)] = jnp.where(
            u >= RBLK // 2, base - (RBLK - 1), base)
        return c
      lax.fori_loop(0, n16, rbody, 0)

    remap(tidx, PAIRS_W // 16)
    remap(cidx, PAIRS_W // 16)
    remap(nidx, NEG_W // 16)

    lane = lax.iota(jnp.int32, 16)

    def issue(ci, trows, crows, nrows, sem):
      cp = ci * CHUNK
      pltpu.async_copy(ttab_hbm.at[tidx.at[pl.ds(cp, CHUNK)]], trows, sem)
      pltpu.async_copy(ctab_hbm.at[cidx.at[pl.ds(cp, CHUNK)]], crows, sem)
      for g in range(NEG_GATHERS):
        pltpu.async_copy(
            ttab_hbm.at[nidx.at[pl.ds(ci * NEG_CHUNK + g * GROWS, GROWS)]],
            nrows.at[pl.ds(g * GROWS, GROWS)], sem)

    def drain(trows, crows, nrows, sem):
      pltpu.make_async_copy(ttab_hbm.at[tidx.at[pl.ds(0, CHUNK)]],
                            trows, sem).wait()
      pltpu.make_async_copy(ctab_hbm.at[cidx.at[pl.ds(0, CHUNK)]],
                            crows, sem).wait()
      for g in range(NEG_GATHERS):
        pltpu.make_async_copy(
            ttab_hbm.at[nidx.at[pl.ds(g * GROWS, GROWS)]],
            nrows.at[pl.ds(g * GROWS, GROWS)], sem).wait()

    def compute(ci, trows, crows, nrows):
      cp = ci * CHUNK
      for grp in range(2):
        p0 = grp * 16
        prow = p0 + lane
        nrow0 = prow * NUM_NEG

        def dbody(d, accs):
          dcol = (d + lane) & (DIM - 1)
          cv = plsc.load_gather(crows, [prow, dcol])
          tv = plsc.load_gather(trows, [prow, dcol])
          outs = [accs[0] + cv * tv]
          for kk in range(NUM_NEG):
            nv = plsc.load_gather(nrows, [nrow0 + kk, dcol])
            outs.append(accs[kk + 1] + nv * cv)
          return tuple(outs)

        accs = lax.fori_loop(
            0, DIM, dbody,
            tuple(jnp.zeros((16,), jnp.float32) for _ in range(NUM_NEG + 1)))
        posv[pl.ds(cp + p0, 16)] = accs[0]
        sbase = (ci * 2 + grp) * (16 * NUM_NEG)
        for kk in range(NUM_NEG):
          negv[pl.ds(sbase + kk * 16, 16)] = -accs[kk + 1]

    issue(0, trowsA, crowsA, nrowsA, semA)

    def pair_body(ci2, carry):
      ca = ci2 * 2
      issue(ca + 1, trowsB, crowsB, nrowsB, semB)
      drain(trowsA, crowsA, nrowsA, semA)
      compute(ca, trowsA, crowsA, nrowsA)

      @pl.when(ci2 < NCHUNK // 2 - 1)
      def _():
        issue(ca + 2, trowsA, crowsA, nrowsA, semA)

      drain(trowsB, crowsB, nrowsB, semB)
      compute(ca + 1, trowsB, crowsB, nrowsB)
      return carry

    lax.fori_loop(0, NCHUNK // 2, pair_body, 0)
    pltpu.sync_copy(posv, pos_hbm.at[pl.ds(pbase, PAIRS_W)])
    pltpu.sync_copy(negv, neg_hbm.at[pl.ds(nbase, NEG_W)])

  return k(tw, cw, ns, ttab, ctab)


RBLK = 32768
NRBLK = (VOCAB + RBLK - 1) // RBLK
VROWS = NRBLK * RBLK


def _transpose_body(in_ref, out_ref):
  x = in_ref[...]
  xcat = jnp.concatenate([x[:, :RBLK // 2], x[:, RBLK // 2:]], axis=0)
  out_ref[...] = jnp.transpose(xcat)


def _relayout(table):
  tv = jnp.swapaxes(table, 0, 1)
  out = pl.pallas_call(
      _transpose_body,
      grid=(NRBLK,),
      in_specs=[pl.BlockSpec((DIM, RBLK), lambda i: (0, i))],
      out_specs=pl.BlockSpec((RBLK // 2, 2 * DIM), lambda i: (i, 0)),
      out_shape=jax.ShapeDtypeStruct((VROWS // 2, 2 * DIM), jnp.float32),
  )(tv)
  return out.reshape(VROWS, DIM)


def _loss_body(pos_ref, neg_ref, out_ref):
  s = jnp.sum(jax.nn.log_sigmoid(pos_ref[...]))
  s = s + jnp.sum(jax.nn.log_sigmoid(neg_ref[...]))
  out_ref[0, 0] = -(s / BATCH)


def kernel(target_words, context_words, negative_samples, target_table,
           context_table):
  tw = target_words.astype(jnp.int32)
  cw = context_words.astype(jnp.int32)
  ns = negative_samples.astype(jnp.int32).reshape(-1)
  pos, neg = _sc_scores(tw, cw, ns, _relayout(target_table),
                        _relayout(context_table))
  out = pl.pallas_call(
      _loss_body,
      out_shape=jax.ShapeDtypeStruct((1, 1), jnp.float32),
      out_specs=pl.BlockSpec(memory_space=pltpu.SMEM),
  )(pos.reshape(128, 128), neg.reshape(BATCH * NUM_NEG // 128, 128))
  return out[0, 0]
